# baseline (speedup 1.0000x reference)
import functools

import jax
import jax.numpy as jnp
from jax import lax
from jax.experimental import pallas as pl
from jax.experimental.pallas import tpu as pltpu
from jax.experimental.pallas import tpu_sc as plsc

_N_CHANNELS = 100
_RESOLUTIONS = (16, 64, 256, 1024)
_OFFSETS = (0, 1700, 8200, 33900)
_SC_RES = ((256, 8200), (1024, 33900))
_TC_RES = ((16, 0, 32), (64, 1700, 64))
_DIM = 64
_BATCH = 4096
_NC, _NS, _L = 2, 16, 16
_NW = _NC * _NS
_BPW = _BATCH // _NW
_NSC = _N_CHANNELS * len(_SC_RES)
_MB = _BPW // _L
_KC = 8


def _bucket_ids(v, r, one, zero):
    scale = jnp.float32(r - 1)
    delta = jnp.float32(1.0) / jnp.float32(r - 1)
    t = v * scale
    f = t.astype(jnp.int32)
    k = f + jnp.where(t > f.astype(jnp.float32), one, zero)
    k = jnp.minimum(k, r - 1)
    km1 = k - 1
    b_lo = jnp.where(km1 < 0, jnp.float32(-1.0),
                     km1.astype(jnp.float32) * delta)
    b_hi = jnp.where(k == r - 1, jnp.float32(1.0),
                     k.astype(jnp.float32) * delta)
    return (km1 + jnp.where(b_lo < v, one, zero)
            + jnp.where(b_hi < v, one, zero))


_MESH = plsc.VectorSubcoreMesh(core_axis_name="c", subcore_axis_name="s",
                               num_cores=_NC, num_subcores=_NS)


@functools.partial(
    pl.kernel,
    out_type=jax.ShapeDtypeStruct((_BATCH, _DIM), jnp.float32),
    mesh=_MESH,
    compiler_params=pltpu.CompilerParams(use_tc_tiling_on_sc=False,
                                         needs_layout_passes=False),
    scratch_types=[
        pltpu.VMEM((_BPW, _N_CHANNELS), jnp.float32),
        pltpu.VMEM((_NSC, _BPW), jnp.int32),
        pltpu.VMEM((_BPW, _DIM), jnp.float32),
        pltpu.SemaphoreType.DMA,
        pltpu.SemaphoreType.DMA,
    ],
)
def _emb_sc(ft_hbm, w_hbm, out_hbm, feat_v, idx_v, acc_v, sem_f, sem0):
    wid = lax.axis_index("s") * _NC + lax.axis_index("c")
    base = wid * _BPW

    pltpu.async_copy(ft_hbm.at[pl.ds(base, _BPW), :], feat_v, sem_f).wait()

    zeros = jnp.zeros((_L,), jnp.float32)

    @pl.loop(0, _BPW, unroll=8)
    def _phase_zero(b):
        for q in range(_DIM // _L):
            acc_v[b, pl.ds(q * _L, _L)] = zeros

    def _fire(j):
        pltpu.async_copy(w_hbm.at[idx_v.at[j]], acc_v, sem0, add=True)

    def _wait_one():
        pltpu.make_async_copy(w_hbm.at[idx_v.at[0]], acc_v, sem0).wait()

    one = jnp.full((_L,), 1, jnp.int32)
    zero = jnp.full((_L,), 0, jnp.int32)
    iota = lax.iota(jnp.int32, _L)
    rows = [m * _L + iota for m in range(_MB)]

    @pl.loop(0, _N_CHANNELS)
    def _phase_main(c):
        cvec = zero + c
        for i, (r, go) in enumerate(_SC_RES):
            off = c * (r + 1) + go
            row = i * _N_CHANNELS + c
            for m in range(_MB):
                v = plsc.load_gather(feat_v, [rows[m], cvec])
                idx_v[row, pl.ds(m * _L, _L)] = (
                    _bucket_ids(v, r, one, zero) + off)

        @pl.when(c >= _KC)
        def _():
            for _i in range(len(_SC_RES)):
                _wait_one()

        for i in range(len(_SC_RES)):
            _fire(i * _N_CHANNELS + c)

    @pl.loop(0, _KC * len(_SC_RES))
    def _phase_drain(_j):
        _wait_one()

    pltpu.sync_copy(acc_v, out_hbm.at[pl.ds(base, _BPW), :])


_TC_BLK = 128


def _tc_body(feat_ref, w16_ref, w64_ref, out_ref):
    fb = feat_ref[...]
    one = jnp.full(fb.shape, 1, jnp.int32)
    zero = jnp.full(fb.shape, 0, jnp.int32)

    acc = jnp.zeros((_TC_BLK, _DIM), jnp.float32)
    for (r, _go, p), w_ref in zip(_TC_RES, (w16_ref, w64_ref)):
        kcol = lax.broadcasted_iota(jnp.int32, (1, 1, p), 2)
        a = _bucket_ids(fb, r, one, zero)
        oh = jnp.where(a[:, :, None] == kcol,
                       jnp.float32(1.0), jnp.float32(0.0))
        oh = oh.astype(jnp.bfloat16).reshape(_TC_BLK, _N_CHANNELS * p)
        w = w_ref[...].reshape(_N_CHANNELS * p, _DIM)
        acc = acc + jax.lax.dot_general(
            oh, w, (((1,), (0,)), ((), ())),
            preferred_element_type=jnp.float32)
    out_ref[...] = acc


_tc_onehot = pl.pallas_call(
    _tc_body,
    grid=(_BATCH // _TC_BLK,),
    in_specs=[
        pl.BlockSpec((_TC_BLK, _N_CHANNELS), lambda i: (i, 0)),
        pl.BlockSpec((_N_CHANNELS, _TC_RES[0][2], _DIM), lambda i: (0, 0, 0)),
        pl.BlockSpec((_N_CHANNELS, _TC_RES[1][2], _DIM), lambda i: (0, 0, 0)),
    ],
    out_specs=pl.BlockSpec((_TC_BLK, _DIM), lambda i: (i, 0)),
    out_shape=jax.ShapeDtypeStruct((_BATCH, _DIM), jnp.float32),
)


def _tc_table(W, r, go, p):
    w = W[go:go + _N_CHANNELS * (r + 1)].astype(jnp.bfloat16)
    w = w.reshape(_N_CHANNELS, r + 1, _DIM)[:, :min(r + 1, p), :]
    if w.shape[1] < p:
        w = jnp.pad(w, ((0, 0), (0, p - w.shape[1]), (0, 0)))
    return w


def kernel(features, W):
    w16 = _tc_table(W, *_TC_RES[0])
    w64 = _tc_table(W, *_TC_RES[1])
    out_sc = _emb_sc(features, W)
    out_tc = _tc_onehot(features, w16, w64)
    return out_sc + out_tc


# speedup vs baseline: 1.1367x; 1.1367x over previous
import functools

import jax
import jax.numpy as jnp
from jax import lax
from jax.experimental import pallas as pl
from jax.experimental.pallas import tpu as pltpu
from jax.experimental.pallas import tpu_sc as plsc

_N_CHANNELS = 100
_RESOLUTIONS = (16, 64, 256, 1024)
_OFFSETS = (0, 1700, 8200, 33900)
_SC_RES = ((64, 1700), (256, 8200), (1024, 33900))
_TC_RES = ((16, 0, 32),)
_FPAD = 128
_DIM = 64
_BATCH = 4096
_NC, _NS, _L = 2, 16, 16
_NW = _NC * _NS
_BPW = _BATCH // _NW
_NSC = _N_CHANNELS * len(_SC_RES)
_MB = _BPW // _L
_KC = 8


def _bucket_ids(v, r, one, zero):
    scale = jnp.float32(r - 1)
    delta = jnp.float32(1.0) / jnp.float32(r - 1)
    t = v * scale
    f = t.astype(jnp.int32)
    k = f + jnp.where(t > f.astype(jnp.float32), one, zero)
    k = jnp.minimum(k, r - 1)
    km1 = k - 1
    b_lo = jnp.where(km1 < 0, jnp.float32(-1.0),
                     km1.astype(jnp.float32) * delta)
    b_hi = jnp.where(k == r - 1, jnp.float32(1.0),
                     k.astype(jnp.float32) * delta)
    return (km1 + jnp.where(b_lo < v, one, zero)
            + jnp.where(b_hi < v, one, zero))


_MESH = plsc.VectorSubcoreMesh(core_axis_name="c", subcore_axis_name="s",
                               num_cores=_NC, num_subcores=_NS)


@functools.partial(
    pl.kernel,
    out_type=jax.ShapeDtypeStruct((_BATCH, _DIM), jnp.float32),
    mesh=_MESH,
    compiler_params=pltpu.CompilerParams(use_tc_tiling_on_sc=False,
                                         needs_layout_passes=False),
    scratch_types=[
        pltpu.VMEM((_BPW, _FPAD), jnp.float32),
        pltpu.VMEM((_NSC, _BPW), jnp.int32),
        pltpu.VMEM((_BPW, _DIM), jnp.float32),
        pltpu.SemaphoreType.DMA,
        pltpu.SemaphoreType.DMA,
    ],
)
def _emb_sc(ft_hbm, w_hbm, out_hbm, feat_v, idx_v, acc_v, sem_f, sem0):
    wid = lax.axis_index("s") * _NC + lax.axis_index("c")
    base = wid * _BPW

    pltpu.async_copy(ft_hbm.at[pl.ds(base, _BPW), :], feat_v, sem_f).wait()

    zeros = jnp.zeros((_L,), jnp.float32)

    @pl.loop(0, _BPW, unroll=8)
    def _phase_zero(b):
        for q in range(_DIM // _L):
            acc_v[b, pl.ds(q * _L, _L)] = zeros

    def _fire(j):
        pltpu.async_copy(w_hbm.at[idx_v.at[j]], acc_v, sem0, add=True)

    def _wait_one():
        pltpu.make_async_copy(w_hbm.at[idx_v.at[0]], acc_v, sem0).wait()

    one = jnp.full((_L,), 1, jnp.int32)
    zero = jnp.full((_L,), 0, jnp.int32)
    iota = lax.iota(jnp.int32, _L)
    rows = [m * _L + iota for m in range(_MB)]

    @pl.loop(0, _N_CHANNELS)
    def _phase_main(c):
        cvec = zero + c
        for i, (r, go) in enumerate(_SC_RES):
            off = c * (r + 1) + go
            row = i * _N_CHANNELS + c
            for m in range(_MB):
                v = plsc.load_gather(feat_v, [rows[m], cvec])
                idx_v[row, pl.ds(m * _L, _L)] = (
                    _bucket_ids(v, r, one, zero) + off)

        @pl.when(c >= _KC)
        def _():
            for _i in range(len(_SC_RES)):
                _wait_one()

        for i in range(len(_SC_RES)):
            _fire(i * _N_CHANNELS + c)

    @pl.loop(0, _KC * len(_SC_RES))
    def _phase_drain(_j):
        _wait_one()

    pltpu.sync_copy(acc_v, out_hbm.at[pl.ds(base, _BPW), :])


_TC_BLK = 128


def _tc_body(feat_ref, w16_ref, out_ref):
    fb = feat_ref[...]
    one = jnp.full(fb.shape, 1, jnp.int32)
    zero = jnp.full(fb.shape, 0, jnp.int32)

    acc = jnp.zeros((_TC_BLK, _DIM), jnp.float32)
    for (r, _go, p), w_ref in zip(_TC_RES, (w16_ref,)):
        kcol = lax.broadcasted_iota(jnp.int32, (1, 1, p), 2)
        a = _bucket_ids(fb, r, one, zero)
        oh = jnp.where(a[:, :, None] == kcol,
                       jnp.float32(1.0), jnp.float32(0.0))
        oh = oh.astype(jnp.bfloat16).reshape(_TC_BLK, _N_CHANNELS * p)
        w = w_ref[...].reshape(_N_CHANNELS * p, _DIM)
        acc = acc + jax.lax.dot_general(
            oh, w, (((1,), (0,)), ((), ())),
            preferred_element_type=jnp.float32)
    out_ref[...] = acc


_tc_onehot = pl.pallas_call(
    _tc_body,
    grid=(_BATCH // _TC_BLK,),
    in_specs=[
        pl.BlockSpec((_TC_BLK, _N_CHANNELS), lambda i: (i, 0)),
        pl.BlockSpec((_N_CHANNELS, _TC_RES[0][2], _DIM), lambda i: (0, 0, 0)),
    ],
    out_specs=pl.BlockSpec((_TC_BLK, _DIM), lambda i: (i, 0)),
    out_shape=jax.ShapeDtypeStruct((_BATCH, _DIM), jnp.float32),
)


def _tc_table(W, r, go, p):
    w = W[go:go + _N_CHANNELS * (r + 1)].astype(jnp.bfloat16)
    w = w.reshape(_N_CHANNELS, r + 1, _DIM)[:, :min(r + 1, p), :]
    if w.shape[1] < p:
        w = jnp.pad(w, ((0, 0), (0, p - w.shape[1]), (0, 0)))
    return w


def kernel(features, W):
    w16 = _tc_table(W, *_TC_RES[0])
    fpad = jnp.pad(features, ((0, 0), (0, _FPAD - _N_CHANNELS)))
    out_sc = _emb_sc(fpad, W)
    out_tc = _tc_onehot(features, w16)
    return out_sc + out_tc
